# Initial kernel scaffold; baseline (speedup 1.0000x reference)
import jax
import jax.numpy as jnp
from jax.experimental import pallas as pl


def kernel(x, edge_index, edge_attr, batch, bond_edge_index, bond_edge_attr, params):
    raise NotImplementedError("write your pallas kernel here")


# speedup vs baseline: 1.3704x; 1.3704x over previous
import functools

import jax
import jax.numpy as jnp
from jax import lax
from jax.experimental import pallas as pl
from jax.experimental.pallas import tpu as pltpu

_INTERPRET = False

_ATOM_DIMS = [119, 5, 12, 12, 10, 6, 6, 2, 2]
_BOND_DIMS = [8, 9, 5]
_D = 128


def _pc(body, grid, in_specs, out_specs, out_shape):
    return pl.pallas_call(
        body,
        grid=grid,
        in_specs=in_specs,
        out_specs=out_specs,
        out_shape=out_shape,
        interpret=_INTERPRET,
    )


def _full_spec(shape):
    return pl.BlockSpec(shape, lambda *args: tuple(0 for _ in shape))


def _dot(a, b):
    return lax.dot_general(a, b, (((1,), (0,)), ((), ())),
                           preferred_element_type=jnp.float32,
                           precision=lax.Precision.HIGHEST)


def _embed_body(nf, widths, x_ref, tab_ref, o_ref):
    g = x_ref[...]
    offs = [0]
    for w in widths[:-1]:
        offs.append(offs[-1] + w)
    tot = tab_ref.shape[0]
    row = lax.broadcasted_iota(jnp.int32, (1, tot), 1)
    oh = jnp.zeros((g.shape[0], tot), jnp.float32)
    for f in range(nf):
        oh = oh + (g[:, f:f + 1] + offs[f] == row).astype(jnp.float32)
    o_ref[...] = _dot(oh, tab_ref[...])


def _embed_sum_tc(x, table, widths, bn):
    n = x.shape[0]
    nf = x.shape[1]
    tot = table.shape[0]
    body = functools.partial(_embed_body, nf, widths)
    return _pc(
        body, (n // bn,),
        [pl.BlockSpec((bn, nf), lambda i: (i, 0)),
         _full_spec((tot, _D))],
        pl.BlockSpec((bn, _D), lambda i: (i, 0)),
        jax.ShapeDtypeStruct((n, _D), jnp.float32))(x, table)


def _atom_msg_body(eh_ref, gat_ref, w_ref, b_ref, o_ref):
    ea = _dot(eh_ref[...], w_ref[...]) + b_ref[...]
    o_ref[...] = jnp.maximum(gat_ref[...] + ea, 0.0)


def _atom_msg(eh, gathered, w, b, bn):
    e = eh.shape[0]
    return _pc(
        _atom_msg_body, (e // bn,),
        [pl.BlockSpec((bn, _D), lambda i: (i, 0)),
         pl.BlockSpec((bn, _D), lambda i: (i, 0)),
         _full_spec((_D, _D)), _full_spec((1, _D))],
        pl.BlockSpec((bn, _D), lambda i: (i, 0)),
        jax.ShapeDtypeStruct((e, _D), jnp.float32))(eh, gathered, w, b)


def _bond_msg_body(gat_ref, ba_ref, aw1_ref, ab1_ref, aw2_ref, ab2_ref,
                   lw_ref, lb_ref, o_ref):
    ca0 = jnp.maximum(ba_ref[...] * aw1_ref[...] + ab1_ref[...], 0.0)
    ca = _dot(ca0, aw2_ref[...]) + ab2_ref[...]
    ea = _dot(ca, lw_ref[...]) + lb_ref[...]
    o_ref[...] = jnp.maximum(gat_ref[...] + ea, 0.0)


def _bond_msg(gathered, battr, aw1, ab1, aw2, ab2, lw, lb, bn):
    e = gathered.shape[0]
    return _pc(
        _bond_msg_body, (e // bn,),
        [pl.BlockSpec((bn, _D), lambda i: (i, 0)),
         pl.BlockSpec((bn, 1), lambda i: (i, 0)),
         _full_spec((1, _D)), _full_spec((1, _D)),
         _full_spec((_D, _D)), _full_spec((1, _D)),
         _full_spec((_D, _D)), _full_spec((1, _D))],
        pl.BlockSpec((bn, _D), lambda i: (i, 0)),
        jax.ShapeDtypeStruct((e, _D), jnp.float32))(
            gathered, battr, aw1, ab1, aw2, ab2, lw, lb)


def _ln(u, g, b):
    mu = jnp.mean(u, axis=1, keepdims=True)
    var = jnp.mean((u - mu) ** 2, axis=1, keepdims=True)
    return (u - mu) * lax.rsqrt(var + 1e-5) * g + b


def _atom_update_body(do_relu, h_ref, a0_ref, a1_ref, w1_ref, b1_ref,
                      w2_ref, b2_ref, lng_ref, lnb_ref, gnw_ref, gnb_ref,
                      gna_ref, o_ref):
    h = h_ref[...]
    t = h + a0_ref[...] + a1_ref[...]
    u = _dot(jnp.maximum(_dot(t, w1_ref[...]) + b1_ref[...], 0.0),
             w2_ref[...]) + b2_ref[...]
    l = _ln(u, lng_ref[...], lnb_ref[...])
    mu0 = jnp.mean(l, axis=0, keepdims=True)
    sub = l - mu0 * gna_ref[...]
    var0 = jnp.mean(sub ** 2, axis=0, keepdims=True)
    g = gnw_ref[...] * sub * lax.rsqrt(var0 + 1e-5) + gnb_ref[...]
    if do_relu:
        g = jnp.maximum(g, 0.0)
    o_ref[...] = g + h


def _atom_update(h, a0, a1, cv, ln_p, gn_p, do_relu):
    n = h.shape[0]
    body = functools.partial(_atom_update_body, do_relu)
    args = (h, a0, a1, cv['w1'], cv['b1'].reshape(1, -1), cv['w2'],
            cv['b2'].reshape(1, -1), ln_p['g'].reshape(1, -1),
            ln_p['b'].reshape(1, -1), gn_p['w'].reshape(1, -1),
            gn_p['b'].reshape(1, -1), gn_p['a'].reshape(1, -1))
    return _pc(
        body, (1,),
        [_full_spec(a.shape) for a in args],
        _full_spec((n, _D)),
        jax.ShapeDtypeStruct((n, _D), jnp.float32))(*args)


def _bond_pre_body(ce_ref, ag_ref, w1_ref, b1_ref, w2_ref, b2_ref,
                   lng_ref, lnb_ref, t_ref, s1_ref, s2_ref):
    t = ce_ref[...] + ag_ref[...]
    u = _dot(jnp.maximum(_dot(t, w1_ref[...]) + b1_ref[...], 0.0),
             w2_ref[...]) + b2_ref[...]
    l = _ln(u, lng_ref[...], lnb_ref[...])
    t_ref[...] = l

    @pl.when(pl.program_id(0) == 0)
    def _():
        s1_ref[...] = jnp.zeros_like(s1_ref)
        s2_ref[...] = jnp.zeros_like(s2_ref)

    s1_ref[...] += jnp.sum(l, axis=0, keepdims=True)
    s2_ref[...] += jnp.sum(l * l, axis=0, keepdims=True)


def _bond_pre(ce, aggr, cv, ln_p, bn):
    e = ce.shape[0]
    args = (ce, aggr, cv['w1'], cv['b1'].reshape(1, -1), cv['w2'],
            cv['b2'].reshape(1, -1), ln_p['g'].reshape(1, -1),
            ln_p['b'].reshape(1, -1))
    in_specs = [pl.BlockSpec((bn, _D), lambda i: (i, 0)),
                pl.BlockSpec((bn, _D), lambda i: (i, 0))]
    in_specs += [_full_spec(a.shape) for a in args[2:]]
    return _pc(
        _bond_pre_body, (e // bn,),
        in_specs,
        [pl.BlockSpec((bn, _D), lambda i: (i, 0)),
         _full_spec((1, _D)), _full_spec((1, _D))],
        [jax.ShapeDtypeStruct((e, _D), jnp.float32),
         jax.ShapeDtypeStruct((1, _D), jnp.float32),
         jax.ShapeDtypeStruct((1, _D), jnp.float32)])(*args)


def _bond_post_body(do_relu, ecount, t_ref, ehp_ref, s1_ref, s2_ref,
                    gnw_ref, gnb_ref, gna_ref, o_ref):
    mu = s1_ref[...] / ecount
    ms2 = s2_ref[...] / ecount
    a = gna_ref[...]
    var0 = ms2 - 2.0 * mu * a * mu + (mu * a) ** 2
    t = t_ref[...]
    sub = t - mu * a
    g = gnw_ref[...] * sub * lax.rsqrt(var0 + 1e-5) + gnb_ref[...]
    if do_relu:
        g = jnp.maximum(g, 0.0)
    o_ref[...] = g + ehp_ref[...]


def _bond_post(t, eh_prev, s1, s2, gn_p, do_relu, bn):
    e = t.shape[0]
    body = functools.partial(_bond_post_body, do_relu, float(e))
    args = (t, eh_prev, s1, s2, gn_p['w'].reshape(1, -1),
            gn_p['b'].reshape(1, -1), gn_p['a'].reshape(1, -1))
    in_specs = [pl.BlockSpec((bn, _D), lambda i: (i, 0)),
                pl.BlockSpec((bn, _D), lambda i: (i, 0))]
    in_specs += [_full_spec(a.shape) for a in args[2:]]
    return _pc(
        body, (e // bn,),
        in_specs,
        pl.BlockSpec((bn, _D), lambda i: (i, 0)),
        jax.ShapeDtypeStruct((e, _D), jnp.float32))(*args)


def _pool_body(g_cnt, h_ref, b_ref, w1_ref, b1_ref, w2_ref, b2_ref, o_ref):
    row = lax.broadcasted_iota(jnp.int32, (1, g_cnt), 1)
    p = (b_ref[...] == row).astype(jnp.float32)
    pooled = lax.dot_general(p, h_ref[...], (((0,), (0,)), ((), ())),
                             preferred_element_type=jnp.float32,
                             precision=lax.Precision.HIGHEST)
    cnts = jnp.sum(p, axis=0, keepdims=True)
    gr = pooled / jnp.maximum(cnts, 1.0).T
    z = jnp.maximum(_dot(gr, w1_ref[...]) + b1_ref[...], 0.0)
    o_ref[...] = _dot(z, w2_ref[...]) + b2_ref[...]


def _pool_final(h, batch2d, g_cnt, fc1w, fc1b, fc2w, fc2b):
    n = h.shape[0]
    body = functools.partial(_pool_body, g_cnt)
    args = (h, batch2d, fc1w, fc1b.reshape(1, -1), fc2w, fc2b.reshape(1, -1))
    return _pc(
        body, (1,),
        [_full_spec(a.shape) for a in args],
        _full_spec((g_cnt, _D)),
        jax.ShapeDtypeStruct((g_cnt, _D), jnp.float32))(*args)


def _gather_rows(table, idx):
    return jnp.take(table, idx, axis=0)


def _scatter_add(msg, dst, nseg):
    a = jax.ops.segment_sum(msg, dst, num_segments=nseg)
    return a, jnp.zeros_like(a)


def kernel(x, edge_index, edge_attr, batch, bond_edge_index, bond_edge_attr,
           params):
    n = x.shape[0]
    e = edge_index.shape[1]
    e2 = bond_edge_index.shape[1]
    g_cnt = 128

    atom_tab = jnp.concatenate(params['atom_tables'], axis=0)
    bond_init_tab = jnp.concatenate(params['bond_init_tables'], axis=0)

    h = _embed_sum_tc(x, atom_tab, _ATOM_DIMS, bn=2000)
    eh = _embed_sum_tc(edge_attr, bond_init_tab, _BOND_DIMS, bn=2000)

    src, dst = edge_index[0], edge_index[1]
    bsrc, bdst = bond_edge_index[0], bond_edge_index[1]

    for i, L in enumerate(params['layers']):
        do_relu = (i == 2)
        gathered = _gather_rows(h, src)
        msg = _atom_msg(eh, gathered, L['atom_conv']['lin_w'],
                        L['atom_conv']['lin_b'].reshape(1, -1), bn=2000)
        a0, a1 = _scatter_add(msg, dst, n)
        h_new = _atom_update(h, a0, a1, L['atom_conv'], L['ln_atom'],
                             L['gn_atom'], do_relu)
        bond_tab = jnp.concatenate(L['bond_tables'], axis=0)
        ce = _embed_sum_tc(edge_attr, bond_tab, _BOND_DIMS, bn=2000)
        gathered_b = _gather_rows(ce, bsrc)
        msg_b = _bond_msg(gathered_b, bond_edge_attr, L['aw1'],
                          L['ab1'].reshape(1, -1), L['aw2'],
                          L['ab2'].reshape(1, -1), L['bond_conv']['lin_w'],
                          L['bond_conv']['lin_b'].reshape(1, -1), bn=2000)
        b0, b1 = _scatter_add(msg_b, bdst, e)
        aggr_b = b0 + b1
        t, s1, s2 = _bond_pre(ce, aggr_b, L['bond_conv'], L['ln_bond'],
                              bn=2000)
        eh = _bond_post(t, eh, s1, s2, L['gn_bond'], do_relu, bn=2000)
        h = h_new

    batch2d = batch.reshape(-1, 1)
    return _pool_final(h, batch2d, g_cnt, params['fc1_w'], params['fc1_b'],
                       params['fc2_w'], params['fc2_b'])


# speedup vs baseline: 1.8024x; 1.3152x over previous
import dataclasses
import functools

import jax
import jax.numpy as jnp
from jax import lax
from jax.experimental import pallas as pl
from jax.experimental.pallas import tpu as pltpu
from jax.experimental.pallas import tpu_sc as plsc

_INTERPRET = False
_NC, _NS = 2, 16
_NW = _NC * _NS


def _sc_mesh():
    return plsc.VectorSubcoreMesh(core_axis_name="c", subcore_axis_name="s")


def _sc_params():
    cp = pltpu.CompilerParams()
    if "needs_layout_passes" in pltpu.CompilerParams.__dataclass_fields__:
        cp = dataclasses.replace(cp, needs_layout_passes=False)
    return cp

_ATOM_DIMS = [119, 5, 12, 12, 10, 6, 6, 2, 2]
_BOND_DIMS = [8, 9, 5]
_D = 128


def _pc(body, grid, in_specs, out_specs, out_shape):
    return pl.pallas_call(
        body,
        grid=grid,
        in_specs=in_specs,
        out_specs=out_specs,
        out_shape=out_shape,
        interpret=_INTERPRET,
    )


def _full_spec(shape):
    return pl.BlockSpec(shape, lambda *args: tuple(0 for _ in shape))


def _dot(a, b):
    return lax.dot_general(a, b, (((1,), (0,)), ((), ())),
                           preferred_element_type=jnp.float32,
                           precision=lax.Precision.HIGHEST)


def _embed_body(nf, widths, x_ref, tab_ref, o_ref):
    g = x_ref[...]
    offs = [0]
    for w in widths[:-1]:
        offs.append(offs[-1] + w)
    tot = tab_ref.shape[0]
    row = lax.broadcasted_iota(jnp.int32, (1, tot), 1)
    oh = jnp.zeros((g.shape[0], tot), jnp.float32)
    for f in range(nf):
        oh = oh + (g[:, f:f + 1] + offs[f] == row).astype(jnp.float32)
    o_ref[...] = _dot(oh, tab_ref[...])


def _embed_sum_tc(x, table, widths, bn):
    n = x.shape[0]
    nf = x.shape[1]
    tot = table.shape[0]
    body = functools.partial(_embed_body, nf, widths)
    return _pc(
        body, (n // bn,),
        [pl.BlockSpec((bn, nf), lambda i: (i, 0)),
         _full_spec((tot, _D))],
        pl.BlockSpec((bn, _D), lambda i: (i, 0)),
        jax.ShapeDtypeStruct((n, _D), jnp.float32))(x, table)


def _atom_msg_body(eh_ref, gat_ref, w_ref, b_ref, o_ref):
    ea = _dot(eh_ref[...], w_ref[...]) + b_ref[...]
    o_ref[...] = jnp.maximum(gat_ref[...] + ea, 0.0)


def _atom_msg(eh, gathered, w, b, bn):
    e = eh.shape[0]
    return _pc(
        _atom_msg_body, (e // bn,),
        [pl.BlockSpec((bn, _D), lambda i: (i, 0)),
         pl.BlockSpec((bn, _D), lambda i: (i, 0)),
         _full_spec((_D, _D)), _full_spec((1, _D))],
        pl.BlockSpec((bn, _D), lambda i: (i, 0)),
        jax.ShapeDtypeStruct((e, _D), jnp.float32))(eh, gathered, w, b)


def _bond_msg_body(gat_ref, ba_ref, aw1_ref, ab1_ref, aw2_ref, ab2_ref,
                   lw_ref, lb_ref, o_ref):
    ca0 = jnp.maximum(ba_ref[...] * aw1_ref[...] + ab1_ref[...], 0.0)
    ca = _dot(ca0, aw2_ref[...]) + ab2_ref[...]
    ea = _dot(ca, lw_ref[...]) + lb_ref[...]
    o_ref[...] = jnp.maximum(gat_ref[...] + ea, 0.0)


def _bond_msg(gathered, battr, aw1, ab1, aw2, ab2, lw, lb, bn):
    e = gathered.shape[0]
    return _pc(
        _bond_msg_body, (e // bn,),
        [pl.BlockSpec((bn, _D), lambda i: (i, 0)),
         pl.BlockSpec((bn, 1), lambda i: (i, 0)),
         _full_spec((1, _D)), _full_spec((1, _D)),
         _full_spec((_D, _D)), _full_spec((1, _D)),
         _full_spec((_D, _D)), _full_spec((1, _D))],
        pl.BlockSpec((bn, _D), lambda i: (i, 0)),
        jax.ShapeDtypeStruct((e, _D), jnp.float32))(
            gathered, battr, aw1, ab1, aw2, ab2, lw, lb)


def _ln(u, g, b):
    mu = jnp.mean(u, axis=1, keepdims=True)
    var = jnp.mean((u - mu) ** 2, axis=1, keepdims=True)
    return (u - mu) * lax.rsqrt(var + 1e-5) * g + b


def _atom_update_body(do_relu, h_ref, a0_ref, a1_ref, w1_ref, b1_ref,
                      w2_ref, b2_ref, lng_ref, lnb_ref, gnw_ref, gnb_ref,
                      gna_ref, o_ref):
    h = h_ref[...]
    t = h + a0_ref[...] + a1_ref[...]
    u = _dot(jnp.maximum(_dot(t, w1_ref[...]) + b1_ref[...], 0.0),
             w2_ref[...]) + b2_ref[...]
    l = _ln(u, lng_ref[...], lnb_ref[...])
    mu0 = jnp.mean(l, axis=0, keepdims=True)
    sub = l - mu0 * gna_ref[...]
    var0 = jnp.mean(sub ** 2, axis=0, keepdims=True)
    g = gnw_ref[...] * sub * lax.rsqrt(var0 + 1e-5) + gnb_ref[...]
    if do_relu:
        g = jnp.maximum(g, 0.0)
    o_ref[...] = g + h


def _atom_update(h, acc2, cv, ln_p, gn_p, do_relu):
    n = h.shape[0]
    body = functools.partial(_atom_update_body, do_relu)
    args = (h, acc2, acc2, cv['w1'], cv['b1'].reshape(1, -1), cv['w2'],
            cv['b2'].reshape(1, -1), ln_p['g'].reshape(1, -1),
            ln_p['b'].reshape(1, -1), gn_p['w'].reshape(1, -1),
            gn_p['b'].reshape(1, -1), gn_p['a'].reshape(1, -1))
    in_specs = [_full_spec((n, _D)),
                pl.BlockSpec((n, _D), lambda i: (0, 0)),
                pl.BlockSpec((n, _D), lambda i: (1, 0))]
    in_specs += [_full_spec(a.shape) for a in args[3:]]
    return _pc(
        body, (1,),
        in_specs,
        _full_spec((n, _D)),
        jax.ShapeDtypeStruct((n, _D), jnp.float32))(*args)


def _bond_pre_body(ce_ref, ag_ref, w1_ref, b1_ref, w2_ref, b2_ref,
                   lng_ref, lnb_ref, t_ref, s1_ref, s2_ref):
    t = ce_ref[...] + ag_ref[...]
    u = _dot(jnp.maximum(_dot(t, w1_ref[...]) + b1_ref[...], 0.0),
             w2_ref[...]) + b2_ref[...]
    l = _ln(u, lng_ref[...], lnb_ref[...])
    t_ref[...] = l

    @pl.when(pl.program_id(0) == 0)
    def _():
        s1_ref[...] = jnp.zeros_like(s1_ref)
        s2_ref[...] = jnp.zeros_like(s2_ref)

    s1_ref[...] += jnp.sum(l, axis=0, keepdims=True)
    s2_ref[...] += jnp.sum(l * l, axis=0, keepdims=True)


def _bond_pre(ce, aggr, cv, ln_p, bn):
    e = ce.shape[0]
    args = (ce, aggr, cv['w1'], cv['b1'].reshape(1, -1), cv['w2'],
            cv['b2'].reshape(1, -1), ln_p['g'].reshape(1, -1),
            ln_p['b'].reshape(1, -1))
    in_specs = [pl.BlockSpec((bn, _D), lambda i: (i, 0)),
                pl.BlockSpec((bn, _D), lambda i: (i, 0))]
    in_specs += [_full_spec(a.shape) for a in args[2:]]
    return _pc(
        _bond_pre_body, (e // bn,),
        in_specs,
        [pl.BlockSpec((bn, _D), lambda i: (i, 0)),
         _full_spec((1, _D)), _full_spec((1, _D))],
        [jax.ShapeDtypeStruct((e, _D), jnp.float32),
         jax.ShapeDtypeStruct((1, _D), jnp.float32),
         jax.ShapeDtypeStruct((1, _D), jnp.float32)])(*args)


def _bond_post_body(do_relu, ecount, t_ref, ehp_ref, s1_ref, s2_ref,
                    gnw_ref, gnb_ref, gna_ref, o_ref):
    mu = s1_ref[...] / ecount
    ms2 = s2_ref[...] / ecount
    a = gna_ref[...]
    var0 = ms2 - 2.0 * mu * a * mu + (mu * a) ** 2
    t = t_ref[...]
    sub = t - mu * a
    g = gnw_ref[...] * sub * lax.rsqrt(var0 + 1e-5) + gnb_ref[...]
    if do_relu:
        g = jnp.maximum(g, 0.0)
    o_ref[...] = g + ehp_ref[...]


def _bond_post(t, eh_prev, s1, s2, gn_p, do_relu, bn):
    e = t.shape[0]
    body = functools.partial(_bond_post_body, do_relu, float(e))
    args = (t, eh_prev, s1, s2, gn_p['w'].reshape(1, -1),
            gn_p['b'].reshape(1, -1), gn_p['a'].reshape(1, -1))
    in_specs = [pl.BlockSpec((bn, _D), lambda i: (i, 0)),
                pl.BlockSpec((bn, _D), lambda i: (i, 0))]
    in_specs += [_full_spec(a.shape) for a in args[2:]]
    return _pc(
        body, (e // bn,),
        in_specs,
        pl.BlockSpec((bn, _D), lambda i: (i, 0)),
        jax.ShapeDtypeStruct((e, _D), jnp.float32))(*args)


def _pool_body(g_cnt, h_ref, b_ref, w1_ref, b1_ref, w2_ref, b2_ref, o_ref):
    row = lax.broadcasted_iota(jnp.int32, (1, g_cnt), 1)
    p = (b_ref[...] == row).astype(jnp.float32)
    pooled = lax.dot_general(p, h_ref[...], (((0,), (0,)), ((), ())),
                             preferred_element_type=jnp.float32,
                             precision=lax.Precision.HIGHEST)
    cnts = jnp.sum(p, axis=0, keepdims=True)
    gr = pooled / jnp.maximum(cnts, 1.0).T
    z = jnp.maximum(_dot(gr, w1_ref[...]) + b1_ref[...], 0.0)
    o_ref[...] = _dot(z, w2_ref[...]) + b2_ref[...]


def _pool_final(h, batch2d, g_cnt, fc1w, fc1b, fc2w, fc2b):
    n = h.shape[0]
    body = functools.partial(_pool_body, g_cnt)
    args = (h, batch2d, fc1w, fc1b.reshape(1, -1), fc2w, fc2b.reshape(1, -1))
    return _pc(
        body, (1,),
        [_full_spec(a.shape) for a in args],
        _full_spec((g_cnt, _D)),
        jax.ShapeDtypeStruct((g_cnt, _D), jnp.float32))(*args)


def _sc_gather_body(nwin, table_hbm, idx_hbm, out_hbm, idx_v, rows_v, sem):
    c = lax.axis_index("c")
    s = lax.axis_index("s")
    w = s * _NC + c
    per = nwin // _NW
    rem = nwin - per * _NW
    start = w * per + jnp.minimum(w, rem)
    cnt = per + (w < rem).astype(jnp.int32)

    def step(i, carry):
        base = (start + i) * 128
        pltpu.sync_copy(idx_hbm.at[pl.ds(base, 128)], idx_v)
        pltpu.async_copy(table_hbm.at[idx_v], rows_v, sem).wait()
        pltpu.sync_copy(rows_v, out_hbm.at[pl.ds(base, 128)])
        return carry

    lax.fori_loop(0, cnt, step, 0)


def _sc_gather(table, idx):
    e = idx.shape[0]
    d = table.shape[1]
    nwin = e // 128
    kern = pl.kernel(
        functools.partial(_sc_gather_body, nwin),
        out_type=jax.ShapeDtypeStruct((e, d), jnp.float32),
        mesh=_sc_mesh(),
        compiler_params=_sc_params(),
        scratch_types=[pltpu.VMEM((128,), jnp.int32),
                       pltpu.VMEM((128, d), jnp.float32),
                       pltpu.SemaphoreType.DMA])
    return kern(table, idx)


def _sc_scatter_atom_body(n, nwin, msg_hbm, dst_hbm, zeros_hbm, out_hbm,
                          idx_v, rows_v, acc):
    c = lax.axis_index("c")
    s = lax.axis_index("s")
    slab = 632
    sstart = jnp.minimum(s * slab, n - slab)
    pltpu.sync_copy(zeros_hbm.at[pl.ds(0, slab)],
                    acc.at[pl.ds(sstart, slab)])
    plsc.subcore_barrier()
    half = nwin // _NC
    per = half // _NS
    rem = half - per * _NS
    start = c * half + s * per + jnp.minimum(s, rem)
    cnt = per + (s < rem).astype(jnp.int32)

    def step(i, carry):
        base = (start + i) * 128
        pltpu.sync_copy(dst_hbm.at[pl.ds(base, 128)], idx_v)
        pltpu.sync_copy(msg_hbm.at[pl.ds(base, 128)], rows_v)
        pltpu.sync_copy(rows_v, acc.at[idx_v], add=True)
        return carry

    lax.fori_loop(0, cnt, step, 0)
    plsc.subcore_barrier()
    pltpu.sync_copy(acc.at[pl.ds(sstart, slab)],
                    out_hbm.at[pl.ds(c * n + sstart, slab)])


def _sc_scatter_atom(msg, dst, n, zeros_hbm):
    e = msg.shape[0]
    kern = pl.kernel(
        functools.partial(_sc_scatter_atom_body, n, e // 128),
        out_type=jax.ShapeDtypeStruct((2 * n, _D), jnp.float32),
        mesh=_sc_mesh(),
        compiler_params=_sc_params(),
        scratch_types=[pltpu.VMEM((128,), jnp.int32),
                       pltpu.VMEM((128, _D), jnp.float32),
                       pltpu.VMEM_SHARED((n, _D), jnp.float32)])
    return kern(msg, dst, zeros_hbm)


_BIN = 8192
_NBIN = 40
_CHUNK = 2000


def _sc_scatter_bond_body(e, msg_hbm, dst_hbm, zeros_hbm, out_hbm,
                          dchunk, cid, cdl, idxw, rows_v, acc, gsem):
    c = lax.axis_index("c")
    s = lax.axis_index("s")
    epe = e // _NS
    my0 = s * epe
    slab = _BIN // _NS
    lanes = lax.iota(jnp.int32, 16)

    def fire(w_base):
        for k in range(8):
            idxw[pl.ds(16 * k, 16)] = cdl[pl.ds(w_base + 16 * k, 16)]
        pltpu.async_copy(msg_hbm.at[cid.at[pl.ds(w_base, 128)]],
                         rows_v, gsem).wait()
        pltpu.sync_copy(rows_v, acc.at[idxw], add=True)

    def do_pass(p_i, carry0):
        p = p_i * _NC + c
        lo = p * _BIN
        pltpu.sync_copy(zeros_hbm.at[pl.ds(0, slab)],
                        acc.at[pl.ds(s * slab, slab)])

        @pl.when(s == 0)
        def _():
            pltpu.sync_copy(zeros_hbm.at[pl.ds(0, 8)],
                            acc.at[pl.ds(_BIN, 8)])

        plsc.subcore_barrier()

        def chunk_step(k, lvl):
            pltpu.sync_copy(dst_hbm.at[pl.ds(my0 + k * _CHUNK, _CHUNK)],
                            dchunk)

            def scan(j, off):
                d = dchunk[pl.ds(j * 16, 16)]
                m = (d >> 13) == p
                mi = m.astype(jnp.int32)
                pos = off + jnp.cumsum(mi) - 1
                plsc.store_scatter(cid, [pos],
                                   lanes + (my0 + k * _CHUNK + j * 16),
                                   mask=m)
                plsc.store_scatter(cdl, [pos], d & (_BIN - 1), mask=m)
                return off + jnp.sum(mi)

            lvl = lax.fori_loop(0, _CHUNK // 16, scan, lvl)
            nw = lvl // 128

            def win(w, carry):
                fire(w * 128)
                return carry

            lax.fori_loop(0, nw, win, 0)

            @pl.when(nw > 0)
            def _():
                for k2 in range(8):
                    v = cid[pl.ds(nw * 128 + 16 * k2, 16)]
                    cid[pl.ds(16 * k2, 16)] = v
                    v2 = cdl[pl.ds(nw * 128 + 16 * k2, 16)]
                    cdl[pl.ds(16 * k2, 16)] = v2

            return lvl - nw * 128

        lvl = lax.fori_loop(0, epe // _CHUNK, chunk_step, jnp.int32(0))

        @pl.when(lvl > 0)
        def _():
            padpos = lvl + lanes
            for k in range(8):
                plsc.store_scatter(cid, [padpos + 16 * k], lanes)
                plsc.store_scatter(cdl, [padpos + 16 * k],
                                   _BIN + (lanes & 7))
            fire(0)

        plsc.subcore_barrier()
        fbase = p * _BIN + s * slab

        @pl.when(fbase + slab <= e)
        def _():
            pltpu.sync_copy(acc.at[pl.ds(s * slab, slab)],
                            out_hbm.at[pl.ds(fbase, slab)])

        plsc.subcore_barrier()
        return carry0

    lax.fori_loop(0, _NBIN // _NC, do_pass, 0)


def _sc_scatter_bond(msg, dst, zeros_hbm):
    e = msg.shape[0]
    kern = pl.kernel(
        functools.partial(_sc_scatter_bond_body, e),
        out_type=jax.ShapeDtypeStruct((e, _D), jnp.float32),
        mesh=_sc_mesh(),
        compiler_params=_sc_params(),
        scratch_types=[pltpu.VMEM((_CHUNK,), jnp.int32),
                       pltpu.VMEM((2304,), jnp.int32),
                       pltpu.VMEM((2304,), jnp.int32),
                       pltpu.VMEM((128,), jnp.int32),
                       pltpu.VMEM((128, _D), jnp.float32),
                       pltpu.VMEM_SHARED((_BIN + 8, _D), jnp.float32),
                       pltpu.SemaphoreType.DMA])
    return kern(msg, dst, zeros_hbm)


def kernel(x, edge_index, edge_attr, batch, bond_edge_index, bond_edge_attr,
           params):
    n = x.shape[0]
    e = edge_index.shape[1]
    e2 = bond_edge_index.shape[1]
    g_cnt = 128

    atom_tab = jnp.concatenate(params['atom_tables'], axis=0)
    bond_init_tab = jnp.concatenate(params['bond_init_tables'], axis=0)

    h = _embed_sum_tc(x, atom_tab, _ATOM_DIMS, bn=2000)
    eh = _embed_sum_tc(edge_attr, bond_init_tab, _BOND_DIMS, bn=2000)

    src, dst = edge_index[0], edge_index[1]
    bsrc, bdst = bond_edge_index[0], bond_edge_index[1]
    zeros_hbm = jnp.zeros((1008, _D), jnp.float32)

    for i, L in enumerate(params['layers']):
        do_relu = (i == 2)
        gathered = _sc_gather(h, src)
        msg = _atom_msg(eh, gathered, L['atom_conv']['lin_w'],
                        L['atom_conv']['lin_b'].reshape(1, -1), bn=2000)
        acc2 = _sc_scatter_atom(msg, dst, n, zeros_hbm)
        h_new = _atom_update(h, acc2, L['atom_conv'], L['ln_atom'],
                             L['gn_atom'], do_relu)
        bond_tab = jnp.concatenate(L['bond_tables'], axis=0)
        ce = _embed_sum_tc(edge_attr, bond_tab, _BOND_DIMS, bn=2000)
        gathered_b = _sc_gather(ce, bsrc)
        msg_b = _bond_msg(gathered_b, bond_edge_attr, L['aw1'],
                          L['ab1'].reshape(1, -1), L['aw2'],
                          L['ab2'].reshape(1, -1), L['bond_conv']['lin_w'],
                          L['bond_conv']['lin_b'].reshape(1, -1), bn=2000)
        aggr_b = _sc_scatter_bond(msg_b, bdst, zeros_hbm)
        t, s1, s2 = _bond_pre(ce, aggr_b, L['bond_conv'], L['ln_bond'],
                              bn=2000)
        eh = _bond_post(t, eh, s1, s2, L['gn_bond'], do_relu, bn=2000)
        h = h_new

    batch2d = batch.reshape(-1, 1)
    return _pool_final(h, batch2d, g_cnt, params['fc1_w'], params['fc1_b'],
                       params['fc2_w'], params['fc2_b'])


# speedup vs baseline: 1.8609x; 1.0325x over previous
import dataclasses
import functools

import jax
import jax.numpy as jnp
from jax import lax
from jax.experimental import pallas as pl
from jax.experimental.pallas import tpu as pltpu
from jax.experimental.pallas import tpu_sc as plsc

_INTERPRET = False
_NC, _NS = 2, 16
_NW = _NC * _NS


def _sc_mesh():
    return plsc.VectorSubcoreMesh(core_axis_name="c", subcore_axis_name="s")


def _sc_params():
    cp = pltpu.CompilerParams()
    if "needs_layout_passes" in pltpu.CompilerParams.__dataclass_fields__:
        cp = dataclasses.replace(cp, needs_layout_passes=False)
    return cp

_ATOM_DIMS = [119, 5, 12, 12, 10, 6, 6, 2, 2]
_BOND_DIMS = [8, 9, 5]
_D = 128


def _pc(body, grid, in_specs, out_specs, out_shape):
    return pl.pallas_call(
        body,
        grid=grid,
        in_specs=in_specs,
        out_specs=out_specs,
        out_shape=out_shape,
        interpret=_INTERPRET,
    )


def _full_spec(shape):
    return pl.BlockSpec(shape, lambda *args: tuple(0 for _ in shape))


def _dot(a, b):
    return lax.dot_general(a, b, (((1,), (0,)), ((), ())),
                           preferred_element_type=jnp.float32,
                           precision=lax.Precision.HIGHEST)


def _embed_body(nf, widths, x_ref, tab_ref, o_ref):
    g = x_ref[...]
    offs = [0]
    for w in widths[:-1]:
        offs.append(offs[-1] + w)
    tot = tab_ref.shape[0]
    row = lax.broadcasted_iota(jnp.int32, (1, tot), 1)
    oh = jnp.zeros((g.shape[0], tot), jnp.float32)
    for f in range(nf):
        oh = oh + (g[:, f:f + 1] + offs[f] == row).astype(jnp.float32)
    o_ref[...] = _dot(oh, tab_ref[...])


def _embed_sum_tc(x, table, widths, bn):
    n = x.shape[0]
    nf = x.shape[1]
    tot = table.shape[0]
    body = functools.partial(_embed_body, nf, widths)
    return _pc(
        body, (n // bn,),
        [pl.BlockSpec((bn, nf), lambda i: (i, 0)),
         _full_spec((tot, _D))],
        pl.BlockSpec((bn, _D), lambda i: (i, 0)),
        jax.ShapeDtypeStruct((n, _D), jnp.float32))(x, table)


def _atom_msg_body(eh_ref, gat_ref, w_ref, b_ref, o_ref):
    ea = _dot(eh_ref[...], w_ref[...]) + b_ref[...]
    o_ref[...] = jnp.maximum(gat_ref[...] + ea, 0.0)


def _atom_msg(eh, gathered, w, b, bn):
    e = eh.shape[0]
    return _pc(
        _atom_msg_body, (e // bn,),
        [pl.BlockSpec((bn, _D), lambda i: (i, 0)),
         pl.BlockSpec((bn, _D), lambda i: (i, 0)),
         _full_spec((_D, _D)), _full_spec((1, _D))],
        pl.BlockSpec((bn, _D), lambda i: (i, 0)),
        jax.ShapeDtypeStruct((e, _D), jnp.float32))(eh, gathered, w, b)


def _bond_msg_body(gat_ref, ba_ref, aw1_ref, ab1_ref, aw2_ref, ab2_ref,
                   lw_ref, lb_ref, o_ref):
    ca0 = jnp.maximum(ba_ref[...] * aw1_ref[...] + ab1_ref[...], 0.0)
    ca = _dot(ca0, aw2_ref[...]) + ab2_ref[...]
    ea = _dot(ca, lw_ref[...]) + lb_ref[...]
    o_ref[...] = jnp.maximum(gat_ref[...] + ea, 0.0)


def _bond_msg(gathered, battr, aw1, ab1, aw2, ab2, lw, lb, bn):
    e = gathered.shape[0]
    return _pc(
        _bond_msg_body, (e // bn,),
        [pl.BlockSpec((bn, _D), lambda i: (i, 0)),
         pl.BlockSpec((bn, 1), lambda i: (i, 0)),
         _full_spec((1, _D)), _full_spec((1, _D)),
         _full_spec((_D, _D)), _full_spec((1, _D)),
         _full_spec((_D, _D)), _full_spec((1, _D))],
        pl.BlockSpec((bn, _D), lambda i: (i, 0)),
        jax.ShapeDtypeStruct((e, _D), jnp.float32))(
            gathered, battr, aw1, ab1, aw2, ab2, lw, lb)


def _ln(u, g, b):
    mu = jnp.mean(u, axis=1, keepdims=True)
    var = jnp.mean((u - mu) ** 2, axis=1, keepdims=True)
    return (u - mu) * lax.rsqrt(var + 1e-5) * g + b


def _atom_update_body(do_relu, h_ref, a0_ref, a1_ref, w1_ref, b1_ref,
                      w2_ref, b2_ref, lng_ref, lnb_ref, gnw_ref, gnb_ref,
                      gna_ref, o_ref):
    h = h_ref[...]
    t = h + a0_ref[...] + a1_ref[...]
    u = _dot(jnp.maximum(_dot(t, w1_ref[...]) + b1_ref[...], 0.0),
             w2_ref[...]) + b2_ref[...]
    l = _ln(u, lng_ref[...], lnb_ref[...])
    mu0 = jnp.mean(l, axis=0, keepdims=True)
    sub = l - mu0 * gna_ref[...]
    var0 = jnp.mean(sub ** 2, axis=0, keepdims=True)
    g = gnw_ref[...] * sub * lax.rsqrt(var0 + 1e-5) + gnb_ref[...]
    if do_relu:
        g = jnp.maximum(g, 0.0)
    o_ref[...] = g + h


def _atom_update(h, acc2, cv, ln_p, gn_p, do_relu):
    n = h.shape[0]
    body = functools.partial(_atom_update_body, do_relu)
    args = (h, acc2, acc2, cv['w1'], cv['b1'].reshape(1, -1), cv['w2'],
            cv['b2'].reshape(1, -1), ln_p['g'].reshape(1, -1),
            ln_p['b'].reshape(1, -1), gn_p['w'].reshape(1, -1),
            gn_p['b'].reshape(1, -1), gn_p['a'].reshape(1, -1))
    in_specs = [_full_spec((n, _D)),
                pl.BlockSpec((n, _D), lambda i: (0, 0)),
                pl.BlockSpec((n, _D), lambda i: (1, 0))]
    in_specs += [_full_spec(a.shape) for a in args[3:]]
    return _pc(
        body, (1,),
        in_specs,
        _full_spec((n, _D)),
        jax.ShapeDtypeStruct((n, _D), jnp.float32))(*args)


def _bond_pre_body(ce_ref, ag_ref, w1_ref, b1_ref, w2_ref, b2_ref,
                   lng_ref, lnb_ref, t_ref, s1_ref, s2_ref):
    t = ce_ref[...] + ag_ref[...]
    u = _dot(jnp.maximum(_dot(t, w1_ref[...]) + b1_ref[...], 0.0),
             w2_ref[...]) + b2_ref[...]
    l = _ln(u, lng_ref[...], lnb_ref[...])
    t_ref[...] = l

    @pl.when(pl.program_id(0) == 0)
    def _():
        s1_ref[...] = jnp.zeros_like(s1_ref)
        s2_ref[...] = jnp.zeros_like(s2_ref)

    s1_ref[...] += jnp.sum(l, axis=0, keepdims=True)
    s2_ref[...] += jnp.sum(l * l, axis=0, keepdims=True)


def _bond_pre(ce, aggr, cv, ln_p, bn):
    e = ce.shape[0]
    args = (ce, aggr, cv['w1'], cv['b1'].reshape(1, -1), cv['w2'],
            cv['b2'].reshape(1, -1), ln_p['g'].reshape(1, -1),
            ln_p['b'].reshape(1, -1))
    in_specs = [pl.BlockSpec((bn, _D), lambda i: (i, 0)),
                pl.BlockSpec((bn, _D), lambda i: (i, 0))]
    in_specs += [_full_spec(a.shape) for a in args[2:]]
    return _pc(
        _bond_pre_body, (e // bn,),
        in_specs,
        [pl.BlockSpec((bn, _D), lambda i: (i, 0)),
         _full_spec((1, _D)), _full_spec((1, _D))],
        [jax.ShapeDtypeStruct((e, _D), jnp.float32),
         jax.ShapeDtypeStruct((1, _D), jnp.float32),
         jax.ShapeDtypeStruct((1, _D), jnp.float32)])(*args)


def _bond_post_body(do_relu, ecount, t_ref, ehp_ref, s1_ref, s2_ref,
                    gnw_ref, gnb_ref, gna_ref, o_ref):
    mu = s1_ref[...] / ecount
    ms2 = s2_ref[...] / ecount
    a = gna_ref[...]
    var0 = ms2 - 2.0 * mu * a * mu + (mu * a) ** 2
    t = t_ref[...]
    sub = t - mu * a
    g = gnw_ref[...] * sub * lax.rsqrt(var0 + 1e-5) + gnb_ref[...]
    if do_relu:
        g = jnp.maximum(g, 0.0)
    o_ref[...] = g + ehp_ref[...]


def _bond_post(t, eh_prev, s1, s2, gn_p, do_relu, bn):
    e = t.shape[0]
    body = functools.partial(_bond_post_body, do_relu, float(e))
    args = (t, eh_prev, s1, s2, gn_p['w'].reshape(1, -1),
            gn_p['b'].reshape(1, -1), gn_p['a'].reshape(1, -1))
    in_specs = [pl.BlockSpec((bn, _D), lambda i: (i, 0)),
                pl.BlockSpec((bn, _D), lambda i: (i, 0))]
    in_specs += [_full_spec(a.shape) for a in args[2:]]
    return _pc(
        body, (e // bn,),
        in_specs,
        pl.BlockSpec((bn, _D), lambda i: (i, 0)),
        jax.ShapeDtypeStruct((e, _D), jnp.float32))(*args)


def _pool_body(g_cnt, h_ref, b_ref, w1_ref, b1_ref, w2_ref, b2_ref, o_ref):
    row = lax.broadcasted_iota(jnp.int32, (1, g_cnt), 1)
    p = (b_ref[...] == row).astype(jnp.float32)
    pooled = lax.dot_general(p, h_ref[...], (((0,), (0,)), ((), ())),
                             preferred_element_type=jnp.float32,
                             precision=lax.Precision.HIGHEST)
    cnts = jnp.sum(p, axis=0, keepdims=True)
    gr = pooled / jnp.maximum(cnts, 1.0).T
    z = jnp.maximum(_dot(gr, w1_ref[...]) + b1_ref[...], 0.0)
    o_ref[...] = _dot(z, w2_ref[...]) + b2_ref[...]


def _pool_final(h, batch2d, g_cnt, fc1w, fc1b, fc2w, fc2b):
    n = h.shape[0]
    body = functools.partial(_pool_body, g_cnt)
    args = (h, batch2d, fc1w, fc1b.reshape(1, -1), fc2w, fc2b.reshape(1, -1))
    return _pc(
        body, (1,),
        [_full_spec(a.shape) for a in args],
        _full_spec((g_cnt, _D)),
        jax.ShapeDtypeStruct((g_cnt, _D), jnp.float32))(*args)


def _sc_gather_body(nwin, table_hbm, idx_hbm, out_hbm,
                    i0, i1, i2, i3, r0, r1, r2, r3, s0, s1, s2, s3):
    c = lax.axis_index("c")
    s = lax.axis_index("s")
    w = s * _NC + c
    per = -(-nwin // _NW)
    per = ((per + 3) // 4) * 4
    start = jnp.minimum(w * per, nwin - per)
    idxb = [i0, i1, i2, i3]
    rowb = [r0, r1, r2, r3]
    semb = [s0, s1, s2, s3]

    def quad(q, carry):
        base0 = (start + q * 4) * 128
        hs = []
        for b in range(4):
            pltpu.sync_copy(idx_hbm.at[pl.ds(base0 + b * 128, 128)], idxb[b])
            hs.append(pltpu.async_copy(table_hbm.at[idxb[b]], rowb[b],
                                       semb[b]))
        for b in range(4):
            hs[b].wait()
            pltpu.sync_copy(rowb[b], out_hbm.at[pl.ds(base0 + b * 128, 128)])
        return carry

    lax.fori_loop(0, per // 4, quad, 0)


def _sc_gather(table, idx):
    e = idx.shape[0]
    d = table.shape[1]
    nwin = e // 128
    kern = pl.kernel(
        functools.partial(_sc_gather_body, nwin),
        out_type=jax.ShapeDtypeStruct((e, d), jnp.float32),
        mesh=_sc_mesh(),
        compiler_params=_sc_params(),
        scratch_types=[pltpu.VMEM((128,), jnp.int32)] * 4
        + [pltpu.VMEM((128, d), jnp.float32)] * 4
        + [pltpu.SemaphoreType.DMA] * 4)
    return kern(table, idx)


def _sc_scatter_atom_body(n, nwin, msg_hbm, dst_hbm, zeros_hbm, out_hbm,
                          i0, i1, r0, r1, acc, m0, m1, a0, a1):
    c = lax.axis_index("c")
    s = lax.axis_index("s")
    slab = 632
    sstart = jnp.minimum(s * slab, n - slab)
    pltpu.sync_copy(zeros_hbm.at[pl.ds(0, slab)],
                    acc.at[pl.ds(sstart, slab)])
    plsc.subcore_barrier()
    half = nwin // _NC
    per = half // _NS
    rem = half - per * _NS
    start = c * half + s * per + jnp.minimum(s, rem)
    cnt = per + (s < rem).astype(jnp.int32)
    idxb = [i0, i1]
    rowb = [r0, r1]
    msem = [m0, m1]
    asem = [a0, a1]

    def pair(q, carry):
        for b in range(2):
            iw = q * 2 + b

            @pl.when(iw < cnt)
            def _(b=b, iw=iw):
                base = (start + iw) * 128
                pltpu.sync_copy(dst_hbm.at[pl.ds(base, 128)], idxb[b])
                pltpu.async_copy(msg_hbm.at[pl.ds(base, 128)],
                                 rowb[b], msem[b])

        for b in range(2):
            iw = q * 2 + b

            @pl.when(iw < cnt)
            def _(b=b, iw=iw):
                base = (start + iw) * 128
                pltpu.make_async_copy(msg_hbm.at[pl.ds(base, 128)],
                                      rowb[b], msem[b]).wait()
                pltpu.async_copy(rowb[b], acc.at[idxb[b]], asem[b],
                                 add=True)

        for b in range(2):
            iw = q * 2 + b

            @pl.when(iw < cnt)
            def _(b=b, iw=iw):
                pltpu.make_async_copy(rowb[b], acc.at[idxb[b]],
                                      asem[b]).wait()

        return carry

    lax.fori_loop(0, (cnt + 1) // 2, pair, 0)
    plsc.subcore_barrier()
    pltpu.sync_copy(acc.at[pl.ds(sstart, slab)],
                    out_hbm.at[pl.ds(c * n + sstart, slab)])


def _sc_scatter_atom(msg, dst, n, zeros_hbm):
    e = msg.shape[0]
    kern = pl.kernel(
        functools.partial(_sc_scatter_atom_body, n, e // 128),
        out_type=jax.ShapeDtypeStruct((2 * n, _D), jnp.float32),
        mesh=_sc_mesh(),
        compiler_params=_sc_params(),
        scratch_types=[pltpu.VMEM((128,), jnp.int32)] * 2
        + [pltpu.VMEM((128, _D), jnp.float32)] * 2
        + [pltpu.VMEM_SHARED((n, _D), jnp.float32)]
        + [pltpu.SemaphoreType.DMA] * 4)
    return kern(msg, dst, zeros_hbm)


_BIN = 8192
_NBIN = 40
_CHUNK = 2000


def _sc_scatter_bond_body(e, msg_hbm, dst_hbm, zeros_hbm, out_hbm,
                          dchA, dchB, cid, cdl, idxw, rows_v, acc, gsem,
                          pA, pB):
    c = lax.axis_index("c")
    s = lax.axis_index("s")
    epe = e // _NS
    my0 = s * epe
    slab = _BIN // _NS
    lanes = lax.iota(jnp.int32, 16)

    def fire(w_base):
        for k in range(8):
            idxw[pl.ds(16 * k, 16)] = cdl[pl.ds(w_base + 16 * k, 16)]
        pltpu.async_copy(msg_hbm.at[cid.at[pl.ds(w_base, 128)]],
                         rows_v, gsem).wait()
        pltpu.sync_copy(rows_v, acc.at[idxw], add=True)

    def do_pass(p_i, carry0):
        p = p_i * _NC + c
        lo = p * _BIN
        pltpu.sync_copy(zeros_hbm.at[pl.ds(0, slab)],
                        acc.at[pl.ds(s * slab, slab)])

        @pl.when(s == 0)
        def _():
            pltpu.sync_copy(zeros_hbm.at[pl.ds(0, 8)],
                            acc.at[pl.ds(_BIN, 8)])

        plsc.subcore_barrier()

        def process(dchunk, k, lvl):
            def scan(j, off):
                d = dchunk[pl.ds(j * 16, 16)]
                m = (d >> 13) == p
                mi = m.astype(jnp.int32)
                pos = off + jnp.cumsum(mi) - 1
                plsc.store_scatter(cid, [pos],
                                   lanes + (my0 + k * _CHUNK + j * 16),
                                   mask=m)
                plsc.store_scatter(cdl, [pos], d & (_BIN - 1), mask=m)
                return off + jnp.sum(mi)

            lvl = lax.fori_loop(0, _CHUNK // 16, scan, lvl)
            nw = lvl // 128

            def win(w, carry):
                fire(w * 128)
                return carry

            lax.fori_loop(0, nw, win, 0)

            @pl.when(nw > 0)
            def _():
                for k2 in range(8):
                    v = cid[pl.ds(nw * 128 + 16 * k2, 16)]
                    cid[pl.ds(16 * k2, 16)] = v
                    v2 = cdl[pl.ds(nw * 128 + 16 * k2, 16)]
                    cdl[pl.ds(16 * k2, 16)] = v2

            return lvl - nw * 128

        pltpu.async_copy(dst_hbm.at[pl.ds(my0, _CHUNK)], dchA, pA)

        def pairstep(kp, lvl):
            k0 = kp * 2
            pltpu.make_async_copy(dst_hbm.at[pl.ds(my0, _CHUNK)],
                                  dchA, pA).wait()
            pltpu.async_copy(
                dst_hbm.at[pl.ds(my0 + (k0 + 1) * _CHUNK, _CHUNK)],
                dchB, pB)
            lvl = process(dchA, k0, lvl)
            pltpu.make_async_copy(dst_hbm.at[pl.ds(my0, _CHUNK)],
                                  dchB, pB).wait()

            @pl.when(kp < (epe // _CHUNK) // 2 - 1)
            def _():
                pltpu.async_copy(
                    dst_hbm.at[pl.ds(my0 + (k0 + 2) * _CHUNK, _CHUNK)],
                    dchA, pA)

            lvl = process(dchB, k0 + 1, lvl)
            return lvl

        lvl = lax.fori_loop(0, (epe // _CHUNK) // 2, pairstep, jnp.int32(0))

        @pl.when(lvl > 0)
        def _():
            padpos = lvl + lanes
            for k in range(8):
                plsc.store_scatter(cid, [padpos + 16 * k], lanes)
                plsc.store_scatter(cdl, [padpos + 16 * k],
                                   _BIN + (lanes & 7))
            fire(0)

        plsc.subcore_barrier()
        fbase = p * _BIN + s * slab

        @pl.when(fbase + slab <= e)
        def _():
            pltpu.sync_copy(acc.at[pl.ds(s * slab, slab)],
                            out_hbm.at[pl.ds(fbase, slab)])

        plsc.subcore_barrier()
        return carry0

    lax.fori_loop(0, _NBIN // _NC, do_pass, 0)


def _sc_scatter_bond(msg, dst, zeros_hbm):
    e = msg.shape[0]
    kern = pl.kernel(
        functools.partial(_sc_scatter_bond_body, e),
        out_type=jax.ShapeDtypeStruct((e, _D), jnp.float32),
        mesh=_sc_mesh(),
        compiler_params=_sc_params(),
        scratch_types=[pltpu.VMEM((_CHUNK,), jnp.int32),
                       pltpu.VMEM((_CHUNK,), jnp.int32),
                       pltpu.VMEM((2304,), jnp.int32),
                       pltpu.VMEM((2304,), jnp.int32),
                       pltpu.VMEM((128,), jnp.int32),
                       pltpu.VMEM((128, _D), jnp.float32),
                       pltpu.VMEM_SHARED((_BIN + 8, _D), jnp.float32),
                       pltpu.SemaphoreType.DMA,
                       pltpu.SemaphoreType.DMA,
                       pltpu.SemaphoreType.DMA])
    return kern(msg, dst, zeros_hbm)


def kernel(x, edge_index, edge_attr, batch, bond_edge_index, bond_edge_attr,
           params):
    n = x.shape[0]
    e = edge_index.shape[1]
    e2 = bond_edge_index.shape[1]
    g_cnt = 128

    atom_tab = jnp.concatenate(params['atom_tables'], axis=0)
    bond_init_tab = jnp.concatenate(params['bond_init_tables'], axis=0)

    h = _embed_sum_tc(x, atom_tab, _ATOM_DIMS, bn=2000)
    eh = _embed_sum_tc(edge_attr, bond_init_tab, _BOND_DIMS, bn=2000)

    src, dst = edge_index[0], edge_index[1]
    bsrc, bdst = bond_edge_index[0], bond_edge_index[1]
    zeros_hbm = jnp.zeros((1008, _D), jnp.float32)

    for i, L in enumerate(params['layers']):
        do_relu = (i == 2)
        gathered = _sc_gather(h, src)
        msg = _atom_msg(eh, gathered, L['atom_conv']['lin_w'],
                        L['atom_conv']['lin_b'].reshape(1, -1), bn=2000)
        acc2 = _sc_scatter_atom(msg, dst, n, zeros_hbm)
        h_new = _atom_update(h, acc2, L['atom_conv'], L['ln_atom'],
                             L['gn_atom'], do_relu)
        bond_tab = jnp.concatenate(L['bond_tables'], axis=0)
        ce = _embed_sum_tc(edge_attr, bond_tab, _BOND_DIMS, bn=2000)
        gathered_b = _sc_gather(ce, bsrc)
        msg_b = _bond_msg(gathered_b, bond_edge_attr, L['aw1'],
                          L['ab1'].reshape(1, -1), L['aw2'],
                          L['ab2'].reshape(1, -1), L['bond_conv']['lin_w'],
                          L['bond_conv']['lin_b'].reshape(1, -1), bn=2000)
        aggr_b = _sc_scatter_bond(msg_b, bdst, zeros_hbm)
        t, s1, s2 = _bond_pre(ce, aggr_b, L['bond_conv'], L['ln_bond'],
                              bn=2000)
        eh = _bond_post(t, eh, s1, s2, L['gn_bond'], do_relu, bn=2000)
        h = h_new

    batch2d = batch.reshape(-1, 1)
    return _pool_final(h, batch2d, g_cnt, params['fc1_w'], params['fc1_b'],
                       params['fc2_w'], params['fc2_b'])
